# Initial kernel scaffold; baseline (speedup 1.0000x reference)
import jax
import jax.numpy as jnp
from jax.experimental import pallas as pl


def kernel(x, edge_attr, edge_index, params):
    raise NotImplementedError("write your pallas kernel here")


# speedup vs baseline: 7.5686x; 7.5686x over previous
import functools

import jax
import jax.numpy as jnp
from jax import lax
from jax.experimental import pallas as pl
from jax.experimental.pallas import tpu as pltpu
from jax.experimental.pallas import tpu_sc as plsc

N_NODES = 10000
N_EDGES = 320000
D_NODE = 128
D_EDGE = 16
LATENT = 32
OUT = 3
STEPS = 5
EPS = 1e-5

_NC = 2
_NS = 16
_NW = _NC * _NS
_EW = N_EDGES // _NW
_CH = 1000
_NCH = _EW // _CH

_EP = N_EDGES * LATENT // 128
_BE = 2000


def _sc_mesh():
    return plsc.VectorSubcoreMesh(core_axis_name="c", subcore_axis_name="s",
                                  num_cores=_NC, num_subcores=_NS)


@functools.cache
def _build_gather2():
    @functools.partial(
        pl.kernel,
        out_type=[jax.ShapeDtypeStruct((N_EDGES, LATENT), jnp.float32),
                  jax.ShapeDtypeStruct((N_EDGES, LATENT), jnp.float32)],
        mesh=_sc_mesh(),
        scratch_types=[pltpu.VMEM((_CH,), jnp.int32),
                       pltpu.VMEM((_CH, LATENT), jnp.float32),
                       pltpu.SemaphoreType.DMA],
        compiler_params=pltpu.CompilerParams(use_tc_tiling_on_sc=False),
    )
    def gather2(h_hbm, col_hbm, row_hbm, hc_out, hr_out, idx_v, rows_v, sem):
        wid = lax.axis_index("s") * _NC + lax.axis_index("c")

        def body(j, carry):
            base = wid * _EW + j * _CH
            pltpu.sync_copy(col_hbm.at[pl.ds(base, _CH)], idx_v)
            pltpu.async_copy(h_hbm.at[idx_v], rows_v, sem).wait()
            pltpu.sync_copy(rows_v, hc_out.at[pl.ds(base, _CH)])
            pltpu.sync_copy(row_hbm.at[pl.ds(base, _CH)], idx_v)
            pltpu.async_copy(h_hbm.at[idx_v], rows_v, sem).wait()
            pltpu.sync_copy(rows_v, hr_out.at[pl.ds(base, _CH)])
            return carry

        lax.fori_loop(0, _NCH, body, 0)

    return gather2


def _sc_gather2(h, col, row):
    return _build_gather2()(h, col, row)


@functools.cache
def _build_scatter_add():
    @functools.partial(
        pl.kernel,
        out_type=jax.ShapeDtypeStruct((_NC, N_NODES, LATENT), jnp.float32),
        mesh=_sc_mesh(),
        scratch_types=[pltpu.VMEM((_CH,), jnp.int32),
                       pltpu.VMEM((_CH, LATENT), jnp.float32),
                       pltpu.VMEM_SHARED((N_NODES, LATENT), jnp.float32),
                       pltpu.SemaphoreType.DMA],
        compiler_params=pltpu.CompilerParams(use_tc_tiling_on_sc=False),
    )
    def scatter_add(msg_hbm, col_hbm, zeros_hbm, out_hbm, idx_v, buf_v, aggr_s, sem):
        cid = lax.axis_index("c")
        sid = lax.axis_index("s")
        wid = sid * _NC + cid

        @pl.when(sid == 0)
        def _zero():
            pltpu.sync_copy(zeros_hbm, aggr_s)

        plsc.subcore_barrier()

        def body(j, carry):
            base = wid * _EW + j * _CH
            pltpu.sync_copy(msg_hbm.at[pl.ds(base, _CH)], buf_v)
            pltpu.sync_copy(col_hbm.at[pl.ds(base, _CH)], idx_v)
            pltpu.sync_copy(buf_v, aggr_s.at[idx_v], add=True)
            return carry

        lax.fori_loop(0, _NCH, body, 0)
        plsc.subcore_barrier()

        @pl.when(sid == 0)
        def _flush():
            pltpu.sync_copy(aggr_s, out_hbm.at[cid])

    return scatter_add


def _sc_scatter_add(msg_flat, col, zeros):
    return _build_scatter_add()(msg_flat, col, zeros)


def _edge_body(hc_ref, hr_ref, e_ref, wa_ref, wb_ref, wc_ref, w2_ref,
               b1_ref, b2_ref, g_ref, bt_ref, m_ref, msg_ref, ne_ref):
    hc = hc_ref[...]
    hr = hr_ref[...]
    e = e_ref[...]
    wa = wa_ref[...]
    wb = wb_ref[...]
    wc = wc_ref[...]
    w2 = w2_ref[...]
    b1 = b1_ref[...]
    b2 = b2_ref[...]
    g = g_ref[...]
    bt = bt_ref[...]
    mm = m_ref[...]

    ec = jnp.dot(e, wc, preferred_element_type=jnp.float32)

    def tail(p):
        a = jnp.maximum(p, 0.0)
        b = jnp.maximum(jnp.dot(a, w2, preferred_element_type=jnp.float32) + b2, 0.0)
        mu = jnp.dot(b, mm, preferred_element_type=jnp.float32)
        d = b - mu
        var = jnp.dot(d * d, mm, preferred_element_type=jnp.float32)
        return d * lax.rsqrt(var + EPS) * g + bt

    pm = (jnp.dot(hc, wa, preferred_element_type=jnp.float32)
          + jnp.dot(hr, wb, preferred_element_type=jnp.float32) + ec + b1)
    pe = (jnp.dot(hr, wa, preferred_element_type=jnp.float32)
          + jnp.dot(hc, wb, preferred_element_type=jnp.float32) + ec + b1)
    msg_ref[...] = tail(pm)
    ne_ref[...] = tail(pe) + e


def _edge_call(hc_p, hr_p, e_p, wa, wb, wc, w2, b1, b2, g, bt, mm):
    grid = (_EP // _BE,)
    row_spec = pl.BlockSpec((_BE, 128), lambda i: (i, 0))

    def wspec(a):
        return pl.BlockSpec(a.shape, lambda i, _nd=a.ndim: (0,) * _nd)

    weights = (wa, wb, wc, w2, b1, b2, g, bt, mm)
    return pl.pallas_call(
        _edge_body,
        grid=grid,
        in_specs=[row_spec, row_spec, row_spec] + [wspec(a) for a in weights],
        out_specs=[row_spec, row_spec],
        out_shape=[jax.ShapeDtypeStruct((_EP, 128), jnp.float32),
                   jax.ShapeDtypeStruct((_EP, 128), jnp.float32)],
    )(hc_p, hr_p, e_p, *weights)


def _node_body(a2_ref, h_ref, wna_ref, wnb_ref, w2_ref, b1_ref, b2_ref,
               g_ref, bt_ref, out_ref):
    aggr = a2_ref[0] + a2_ref[1]
    h = h_ref[...]
    pre = (jnp.dot(aggr, wna_ref[...], preferred_element_type=jnp.float32)
           + jnp.dot(h, wnb_ref[...], preferred_element_type=jnp.float32)
           + b1_ref[...])
    a = jnp.maximum(pre, 0.0)
    b = jnp.maximum(jnp.dot(a, w2_ref[...], preferred_element_type=jnp.float32)
                    + b2_ref[...], 0.0)
    mu = jnp.mean(b, axis=-1, keepdims=True)
    d = b - mu
    var = jnp.mean(d * d, axis=-1, keepdims=True)
    out_ref[...] = d * lax.rsqrt(var + EPS) * g_ref[...] + bt_ref[...] + h


def _node_call(aggr2, h, wna, wnb, w2, b1, b2, g, bt):
    return pl.pallas_call(
        _node_body,
        out_shape=jax.ShapeDtypeStruct((N_NODES, LATENT), jnp.float32),
    )(aggr2, h, wna, wnb, w2, b1, b2, g, bt)


def _enc_node_body(x_ref, w1_ref, w2_ref, b1_ref, b2_ref, g_ref, bt_ref, out_ref):
    a = jnp.maximum(jnp.dot(x_ref[...], w1_ref[...],
                            preferred_element_type=jnp.float32) + b1_ref[...], 0.0)
    b = jnp.maximum(jnp.dot(a, w2_ref[...],
                            preferred_element_type=jnp.float32) + b2_ref[...], 0.0)
    mu = jnp.mean(b, axis=-1, keepdims=True)
    d = b - mu
    var = jnp.mean(d * d, axis=-1, keepdims=True)
    out_ref[...] = d * lax.rsqrt(var + EPS) * g_ref[...] + bt_ref[...]


def _enc_node_call(x, w1, w2, b1, b2, g, bt):
    return pl.pallas_call(
        _enc_node_body,
        out_shape=jax.ShapeDtypeStruct((N_NODES, LATENT), jnp.float32),
    )(x, w1, w2, b1, b2, g, bt)


def _enc_edge_body(ea_ref, w1_ref, w2_ref, b1_ref, b2_ref, g_ref, bt_ref,
                   m_ref, out_ref):
    a = jnp.maximum(jnp.dot(ea_ref[...], w1_ref[...],
                            preferred_element_type=jnp.float32) + b1_ref[...], 0.0)
    b = jnp.maximum(jnp.dot(a, w2_ref[...],
                            preferred_element_type=jnp.float32) + b2_ref[...], 0.0)
    mu = jnp.dot(b, m_ref[...], preferred_element_type=jnp.float32)
    d = b - mu
    var = jnp.dot(d * d, m_ref[...], preferred_element_type=jnp.float32)
    out_ref[...] = d * lax.rsqrt(var + EPS) * g_ref[...] + bt_ref[...]


def _enc_edge_call(ea_p, w1, w2, b1, b2, g, bt, mm):
    rows = N_EDGES * D_EDGE // 128
    blk = 2000
    grid = (rows // blk,)
    in_spec = pl.BlockSpec((blk, 128), lambda i: (i, 0))
    out_spec = pl.BlockSpec((blk, 256), lambda i: (i, 0))

    def wspec(a):
        return pl.BlockSpec(a.shape, lambda i, _nd=a.ndim: (0,) * _nd)

    weights = (w1, w2, b1, b2, g, bt, mm)
    return pl.pallas_call(
        _enc_edge_body,
        grid=grid,
        in_specs=[in_spec] + [wspec(a) for a in weights],
        out_specs=out_spec,
        out_shape=jax.ShapeDtypeStruct((rows, 256), jnp.float32),
    )(ea_p, *weights)


def _dec_body(h_ref, w1_ref, w2_ref, b1_ref, b2_ref, out_ref):
    a = jnp.maximum(jnp.dot(h_ref[...], w1_ref[...],
                            preferred_element_type=jnp.float32) + b1_ref[...], 0.0)
    out_ref[...] = jnp.dot(a, w2_ref[...],
                           preferred_element_type=jnp.float32) + b2_ref[...]


def _dec_call(h, w1, w2, b1, b2):
    return pl.pallas_call(
        _dec_body,
        out_shape=jax.ShapeDtypeStruct((N_NODES, OUT), jnp.float32),
    )(h, w1, w2, b1, b2)


def _bd(w, k):
    return jnp.kron(jnp.eye(k, dtype=w.dtype), w)


def kernel(x, edge_attr, edge_index, params):
    row = edge_index[0]
    col = edge_index[1]

    pn = params['node_enc']
    pe = params['edge_enc']
    pm = params['edge_net']
    pv = params['node_net']
    pd = params['decode']

    n_b1 = pn['b1'][None, :]
    n_b2 = pn['b2'][None, :]
    n_g = pn['g'][None, :]
    n_bt = pn['beta'][None, :]

    e_w1 = _bd(pe['W1'], 8)
    e_w2 = _bd(pe['W2'], 8)
    e_b1 = jnp.tile(pe['b1'], 8)[None, :]
    e_b2 = jnp.tile(pe['b2'], 8)[None, :]
    e_g = jnp.tile(pe['g'], 8)[None, :]
    e_bt = jnp.tile(pe['beta'], 8)[None, :]
    m8 = _bd(jnp.full((LATENT, LATENT), 1.0 / LATENT, jnp.float32), 8)

    wa = _bd(pm['W1'][:LATENT], 4)
    wb = _bd(pm['W1'][LATENT:2 * LATENT], 4)
    wc = _bd(pm['W1'][2 * LATENT:], 4)
    w2 = _bd(pm['W2'], 4)
    b1 = jnp.tile(pm['b1'], 4)[None, :]
    b2 = jnp.tile(pm['b2'], 4)[None, :]
    g4 = jnp.tile(pm['g'], 4)[None, :]
    bt4 = jnp.tile(pm['beta'], 4)[None, :]
    m4 = _bd(jnp.full((LATENT, LATENT), 1.0 / LATENT, jnp.float32), 4)

    wna = pv['W1'][:LATENT]
    wnb = pv['W1'][LATENT:]
    v_b1 = pv['b1'][None, :]
    v_b2 = pv['b2'][None, :]
    v_g = pv['g'][None, :]
    v_bt = pv['beta'][None, :]

    d_b1 = pd['b1'][None, :]
    d_b2 = pd['b2'][None, :]

    zeros = jnp.zeros((N_NODES, LATENT), jnp.float32)

    h = _enc_node_call(x, pn['W1'], pn['W2'], n_b1, n_b2, n_g, n_bt)
    ea_p = edge_attr.reshape(N_EDGES * D_EDGE // 128, 128)
    e_p = _enc_edge_call(ea_p, e_w1, e_w2, e_b1, e_b2, e_g, e_bt, m8)
    e_p = e_p.reshape(_EP, 128)

    for _ in range(STEPS):
        hc_flat, hr_flat = _sc_gather2(h, col, row)
        hc_p = hc_flat.reshape(_EP, 128)
        hr_p = hr_flat.reshape(_EP, 128)
        msg_p, newe_p = _edge_call(hc_p, hr_p, e_p,
                                   wa, wb, wc, w2, b1, b2, g4, bt4, m4)
        msg_flat = msg_p.reshape(N_EDGES, LATENT)
        aggr2 = _sc_scatter_add(msg_flat, col, zeros)
        h = _node_call(aggr2, h, wna, wnb, pv['W2'], v_b1, v_b2, v_g, v_bt)
        e_p = newe_p

    return _dec_call(h, pd['W1'], pd['W2'], d_b1, d_b2)


# speedup vs baseline: 8.3385x; 1.1017x over previous
import functools

import jax
import jax.numpy as jnp
from jax import lax
from jax.experimental import pallas as pl
from jax.experimental.pallas import tpu as pltpu
from jax.experimental.pallas import tpu_sc as plsc

N_NODES = 10000
N_EDGES = 320000
D_NODE = 128
D_EDGE = 16
LATENT = 32
OUT = 3
STEPS = 5
EPS = 1e-5

_NC = 2
_NS = 16
_NW = _NC * _NS
_EW = N_EDGES // _NW
_CH = 1000
_NCH = _EW // _CH

_EP = N_EDGES * LATENT // 128
_BE = 2000


def _sc_mesh():
    return plsc.VectorSubcoreMesh(core_axis_name="c", subcore_axis_name="s",
                                  num_cores=_NC, num_subcores=_NS)


@functools.cache
def _build_gather2():
    @functools.partial(
        pl.kernel,
        out_type=[jax.ShapeDtypeStruct((N_EDGES, LATENT), jnp.float32),
                  jax.ShapeDtypeStruct((N_EDGES, LATENT), jnp.float32)],
        mesh=_sc_mesh(),
        scratch_types=[pltpu.VMEM((_NCH, _CH), jnp.int32),
                       pltpu.VMEM((_NCH, _CH), jnp.int32),
                       pltpu.VMEM((_CH, LATENT), jnp.float32),
                       pltpu.VMEM((_CH, LATENT), jnp.float32),
                       pltpu.SemaphoreType.DMA,
                       pltpu.SemaphoreType.DMA,
                       pltpu.SemaphoreType.DMA,
                       pltpu.SemaphoreType.DMA],
        compiler_params=pltpu.CompilerParams(use_tc_tiling_on_sc=False),
    )
    def gather2(h_hbm, col_hbm, row_hbm, hc_out, hr_out,
                cidx_v, ridx_v, buf_a, buf_b, sga, sgb, swa, swb):
        wid = lax.axis_index("s") * _NC + lax.axis_index("c")
        pltpu.sync_copy(col_hbm.at[pl.ds(wid * _NCH, _NCH)], cidx_v)
        pltpu.sync_copy(row_hbm.at[pl.ds(wid * _NCH, _NCH)], ridx_v)

        def ods(j):
            return pl.ds(wid * _EW + j * _CH, _CH)

        pltpu.async_copy(h_hbm.at[cidx_v.at[0]], buf_a, sga)

        def body(j, carry):
            @pl.when(j > 0)
            def _():
                pltpu.make_async_copy(buf_b, hr_out.at[ods(j - 1)], swb).wait()
            pltpu.async_copy(h_hbm.at[ridx_v.at[j]], buf_b, sgb)
            pltpu.make_async_copy(h_hbm.at[cidx_v.at[j]], buf_a, sga).wait()
            pltpu.async_copy(buf_a, hc_out.at[ods(j)], swa)
            pltpu.make_async_copy(buf_a, hc_out.at[ods(j)], swa).wait()

            @pl.when(j < _NCH - 1)
            def _():
                pltpu.async_copy(h_hbm.at[cidx_v.at[j + 1]], buf_a, sga)
            pltpu.make_async_copy(h_hbm.at[ridx_v.at[j]], buf_b, sgb).wait()
            pltpu.async_copy(buf_b, hr_out.at[ods(j)], swb)
            return carry

        lax.fori_loop(0, _NCH, body, 0)
        pltpu.make_async_copy(buf_b, hr_out.at[ods(_NCH - 1)], swb).wait()

    return gather2


def _sc_gather2(h, col2d, row2d):
    return _build_gather2()(h, col2d, row2d)


@functools.cache
def _build_scatter_add():
    @functools.partial(
        pl.kernel,
        out_type=jax.ShapeDtypeStruct((_NC, N_NODES, LATENT), jnp.float32),
        mesh=_sc_mesh(),
        scratch_types=[pltpu.VMEM((_NCH, _CH), jnp.int32),
                       pltpu.VMEM((_CH, LATENT), jnp.float32),
                       pltpu.VMEM((_CH, LATENT), jnp.float32),
                       pltpu.VMEM_SHARED((N_NODES, LATENT), jnp.float32),
                       pltpu.SemaphoreType.DMA,
                       pltpu.SemaphoreType.DMA,
                       pltpu.SemaphoreType.DMA,
                       pltpu.SemaphoreType.DMA],
        compiler_params=pltpu.CompilerParams(use_tc_tiling_on_sc=False),
    )
    def scatter_add(msg_hbm, col_hbm, zeros_hbm, out_hbm,
                    idx_v, buf_a, buf_b, aggr_s, sla, slb, ssa, ssb):
        cid = lax.axis_index("c")
        sid = lax.axis_index("s")
        wid = sid * _NC + cid
        pltpu.sync_copy(col_hbm.at[pl.ds(wid * _NCH, _NCH)], idx_v)

        @pl.when(sid == 0)
        def _zero():
            pltpu.sync_copy(zeros_hbm, aggr_s)

        plsc.subcore_barrier()

        def mds(j):
            return pl.ds(wid * _EW + j * _CH, _CH)

        pltpu.async_copy(msg_hbm.at[mds(0)], buf_a, sla)

        def body(i, carry):
            ka = 2 * i
            kb = 2 * i + 1

            @pl.when(i > 0)
            def _():
                pltpu.make_async_copy(buf_b, aggr_s.at[idx_v.at[kb - 2]], ssb).wait()
            pltpu.async_copy(msg_hbm.at[mds(kb)], buf_b, slb)
            pltpu.make_async_copy(msg_hbm.at[mds(ka)], buf_a, sla).wait()
            pltpu.async_copy(buf_a, aggr_s.at[idx_v.at[ka]], ssa, add=True)
            pltpu.make_async_copy(buf_a, aggr_s.at[idx_v.at[ka]], ssa).wait()

            @pl.when(i < _NCH // 2 - 1)
            def _():
                pltpu.async_copy(msg_hbm.at[mds(ka + 2)], buf_a, sla)
            pltpu.make_async_copy(msg_hbm.at[mds(kb)], buf_b, slb).wait()
            pltpu.async_copy(buf_b, aggr_s.at[idx_v.at[kb]], ssb, add=True)
            return carry

        lax.fori_loop(0, _NCH // 2, body, 0)
        pltpu.make_async_copy(buf_b, aggr_s.at[idx_v.at[_NCH - 1]], ssb).wait()
        plsc.subcore_barrier()

        @pl.when(sid == 0)
        def _flush():
            pltpu.sync_copy(aggr_s, out_hbm.at[cid])

    return scatter_add


def _sc_scatter_add(msg_flat, col2d, zeros):
    return _build_scatter_add()(msg_flat, col2d, zeros)


def _edge_body(hc_ref, hr_ref, e_ref, wa_ref, wb_ref, wc_ref, w2_ref,
               b1_ref, b2_ref, g_ref, bt_ref, m_ref, msg_ref, ne_ref):
    hc = hc_ref[...]
    hr = hr_ref[...]
    e = e_ref[...]
    wa = wa_ref[...]
    wb = wb_ref[...]
    wc = wc_ref[...]
    w2 = w2_ref[...]
    b1 = b1_ref[...]
    b2 = b2_ref[...]
    g = g_ref[...]
    bt = bt_ref[...]
    mm = m_ref[...]

    ec = jnp.dot(e, wc, preferred_element_type=jnp.float32)

    def tail(p):
        a = jnp.maximum(p, 0.0)
        b = jnp.maximum(jnp.dot(a, w2, preferred_element_type=jnp.float32) + b2, 0.0)
        mu = jnp.dot(b, mm, preferred_element_type=jnp.float32)
        d = b - mu
        var = jnp.dot(d * d, mm, preferred_element_type=jnp.float32)
        return d * lax.rsqrt(var + EPS) * g + bt

    pm = (jnp.dot(hc, wa, preferred_element_type=jnp.float32)
          + jnp.dot(hr, wb, preferred_element_type=jnp.float32) + ec + b1)
    pe = (jnp.dot(hr, wa, preferred_element_type=jnp.float32)
          + jnp.dot(hc, wb, preferred_element_type=jnp.float32) + ec + b1)
    msg_ref[...] = tail(pm)
    ne_ref[...] = tail(pe) + e


def _edge_call(hc_p, hr_p, e_p, wa, wb, wc, w2, b1, b2, g, bt, mm):
    grid = (_EP // _BE,)
    row_spec = pl.BlockSpec((_BE, 128), lambda i: (i, 0))

    def wspec(a):
        return pl.BlockSpec(a.shape, lambda i, _nd=a.ndim: (0,) * _nd)

    weights = (wa, wb, wc, w2, b1, b2, g, bt, mm)
    return pl.pallas_call(
        _edge_body,
        grid=grid,
        in_specs=[row_spec, row_spec, row_spec] + [wspec(a) for a in weights],
        out_specs=[row_spec, row_spec],
        out_shape=[jax.ShapeDtypeStruct((_EP, 128), jnp.float32),
                   jax.ShapeDtypeStruct((_EP, 128), jnp.float32)],
    )(hc_p, hr_p, e_p, *weights)


def _node_body(a2_ref, h_ref, wna_ref, wnb_ref, w2_ref, b1_ref, b2_ref,
               g_ref, bt_ref, out_ref):
    aggr = a2_ref[0] + a2_ref[1]
    h = h_ref[...]
    pre = (jnp.dot(aggr, wna_ref[...], preferred_element_type=jnp.float32)
           + jnp.dot(h, wnb_ref[...], preferred_element_type=jnp.float32)
           + b1_ref[...])
    a = jnp.maximum(pre, 0.0)
    b = jnp.maximum(jnp.dot(a, w2_ref[...], preferred_element_type=jnp.float32)
                    + b2_ref[...], 0.0)
    mu = jnp.mean(b, axis=-1, keepdims=True)
    d = b - mu
    var = jnp.mean(d * d, axis=-1, keepdims=True)
    out_ref[...] = d * lax.rsqrt(var + EPS) * g_ref[...] + bt_ref[...] + h


def _node_call(aggr2, h, wna, wnb, w2, b1, b2, g, bt):
    return pl.pallas_call(
        _node_body,
        out_shape=jax.ShapeDtypeStruct((N_NODES, LATENT), jnp.float32),
    )(aggr2, h, wna, wnb, w2, b1, b2, g, bt)


def _enc_node_body(x_ref, w1_ref, w2_ref, b1_ref, b2_ref, g_ref, bt_ref, out_ref):
    a = jnp.maximum(jnp.dot(x_ref[...], w1_ref[...],
                            preferred_element_type=jnp.float32) + b1_ref[...], 0.0)
    b = jnp.maximum(jnp.dot(a, w2_ref[...],
                            preferred_element_type=jnp.float32) + b2_ref[...], 0.0)
    mu = jnp.mean(b, axis=-1, keepdims=True)
    d = b - mu
    var = jnp.mean(d * d, axis=-1, keepdims=True)
    out_ref[...] = d * lax.rsqrt(var + EPS) * g_ref[...] + bt_ref[...]


def _enc_node_call(x, w1, w2, b1, b2, g, bt):
    return pl.pallas_call(
        _enc_node_body,
        out_shape=jax.ShapeDtypeStruct((N_NODES, LATENT), jnp.float32),
    )(x, w1, w2, b1, b2, g, bt)


def _enc_edge_body(ea_ref, w1_ref, w2_ref, b1_ref, b2_ref, g_ref, bt_ref,
                   m_ref, out_ref):
    a = jnp.maximum(jnp.dot(ea_ref[...], w1_ref[...],
                            preferred_element_type=jnp.float32) + b1_ref[...], 0.0)
    b = jnp.maximum(jnp.dot(a, w2_ref[...],
                            preferred_element_type=jnp.float32) + b2_ref[...], 0.0)
    mu = jnp.dot(b, m_ref[...], preferred_element_type=jnp.float32)
    d = b - mu
    var = jnp.dot(d * d, m_ref[...], preferred_element_type=jnp.float32)
    out_ref[...] = d * lax.rsqrt(var + EPS) * g_ref[...] + bt_ref[...]


def _enc_edge_call(ea_p, w1, w2, b1, b2, g, bt, mm):
    rows = N_EDGES * D_EDGE // 128
    blk = 2000
    grid = (rows // blk,)
    in_spec = pl.BlockSpec((blk, 128), lambda i: (i, 0))
    out_spec = pl.BlockSpec((blk, 256), lambda i: (i, 0))

    def wspec(a):
        return pl.BlockSpec(a.shape, lambda i, _nd=a.ndim: (0,) * _nd)

    weights = (w1, w2, b1, b2, g, bt, mm)
    return pl.pallas_call(
        _enc_edge_body,
        grid=grid,
        in_specs=[in_spec] + [wspec(a) for a in weights],
        out_specs=out_spec,
        out_shape=jax.ShapeDtypeStruct((rows, 256), jnp.float32),
    )(ea_p, *weights)


def _dec_body(h_ref, w1_ref, w2_ref, b1_ref, b2_ref, out_ref):
    a = jnp.maximum(jnp.dot(h_ref[...], w1_ref[...],
                            preferred_element_type=jnp.float32) + b1_ref[...], 0.0)
    out_ref[...] = jnp.dot(a, w2_ref[...],
                           preferred_element_type=jnp.float32) + b2_ref[...]


def _dec_call(h, w1, w2, b1, b2):
    return pl.pallas_call(
        _dec_body,
        out_shape=jax.ShapeDtypeStruct((N_NODES, OUT), jnp.float32),
    )(h, w1, w2, b1, b2)


def _bd(w, k):
    return jnp.kron(jnp.eye(k, dtype=w.dtype), w)


def kernel(x, edge_attr, edge_index, params):
    row2d = edge_index[0].reshape(_NW * _NCH, _CH)
    col2d = edge_index[1].reshape(_NW * _NCH, _CH)

    pn = params['node_enc']
    pe = params['edge_enc']
    pm = params['edge_net']
    pv = params['node_net']
    pd = params['decode']

    n_b1 = pn['b1'][None, :]
    n_b2 = pn['b2'][None, :]
    n_g = pn['g'][None, :]
    n_bt = pn['beta'][None, :]

    e_w1 = _bd(pe['W1'], 8)
    e_w2 = _bd(pe['W2'], 8)
    e_b1 = jnp.tile(pe['b1'], 8)[None, :]
    e_b2 = jnp.tile(pe['b2'], 8)[None, :]
    e_g = jnp.tile(pe['g'], 8)[None, :]
    e_bt = jnp.tile(pe['beta'], 8)[None, :]
    m8 = _bd(jnp.full((LATENT, LATENT), 1.0 / LATENT, jnp.float32), 8)

    wa = _bd(pm['W1'][:LATENT], 4)
    wb = _bd(pm['W1'][LATENT:2 * LATENT], 4)
    wc = _bd(pm['W1'][2 * LATENT:], 4)
    w2 = _bd(pm['W2'], 4)
    b1 = jnp.tile(pm['b1'], 4)[None, :]
    b2 = jnp.tile(pm['b2'], 4)[None, :]
    g4 = jnp.tile(pm['g'], 4)[None, :]
    bt4 = jnp.tile(pm['beta'], 4)[None, :]
    m4 = _bd(jnp.full((LATENT, LATENT), 1.0 / LATENT, jnp.float32), 4)

    wna = pv['W1'][:LATENT]
    wnb = pv['W1'][LATENT:]
    v_b1 = pv['b1'][None, :]
    v_b2 = pv['b2'][None, :]
    v_g = pv['g'][None, :]
    v_bt = pv['beta'][None, :]

    d_b1 = pd['b1'][None, :]
    d_b2 = pd['b2'][None, :]

    zeros = jnp.zeros((N_NODES, LATENT), jnp.float32)

    h = _enc_node_call(x, pn['W1'], pn['W2'], n_b1, n_b2, n_g, n_bt)
    ea_p = edge_attr.reshape(N_EDGES * D_EDGE // 128, 128)
    e_p = _enc_edge_call(ea_p, e_w1, e_w2, e_b1, e_b2, e_g, e_bt, m8)
    e_p = e_p.reshape(_EP, 128)

    for _ in range(STEPS):
        hc_flat, hr_flat = _sc_gather2(h, col2d, row2d)
        hc_p = hc_flat.reshape(_EP, 128)
        hr_p = hr_flat.reshape(_EP, 128)
        msg_p, newe_p = _edge_call(hc_p, hr_p, e_p,
                                   wa, wb, wc, w2, b1, b2, g4, bt4, m4)
        msg_flat = msg_p.reshape(N_EDGES, LATENT)
        aggr2 = _sc_scatter_add(msg_flat, col2d, zeros)
        h = _node_call(aggr2, h, wna, wnb, pv['W2'], v_b1, v_b2, v_g, v_bt)
        e_p = newe_p

    return _dec_call(h, pd['W1'], pd['W2'], d_b1, d_b2)
